# Initial kernel scaffold; baseline (speedup 1.0000x reference)
import jax
import jax.numpy as jnp
from jax.experimental import pallas as pl


def kernel(x, edge_index, W1, att_src1, att_dst1, bias1, W2, att_src2, att_dst2, bias2):
    raise NotImplementedError("write your pallas kernel here")


# speedup vs baseline: 33.6147x; 33.6147x over previous
import functools

import jax
import jax.numpy as jnp
from jax import lax
from jax.experimental import pallas as pl
from jax.experimental.pallas import tpu as pltpu
from jax.experimental.pallas import tpu_sc as plsc

N = 10000
NP = 10240
D = 128
H = 8
HC = 64
NC = 40
NCP = 64
L = 16
NCORES = 2
NSUB = 16
NWORK = NCORES * NSUB
ROWS_PER_SUB = NP // NSUB

EPS = 1e-16

_GDN = jax.lax.GatherDimensionNumbers(
    offset_dims=(), collapsed_slice_dims=(0,), start_index_map=(0,))


def _vgather(x, idx):
    return jax.lax.gather(x, idx[:, None], _GDN, (1,),
                          mode=jax.lax.GatherScatterMode.PROMISE_IN_BOUNDS)


def _lane_iota():
    return lax.broadcasted_iota(jnp.int32, (L,), 0)


_mesh = functools.partial(plsc.VectorSubcoreMesh,
                          core_axis_name="core", subcore_axis_name="subcore")

_sc_params = pltpu.CompilerParams(use_tc_tiling_on_sc=False,
                                  needs_layout_passes=False)


def _dot(a, b):
    return jnp.dot(a, b, precision=jax.lax.Precision.HIGHEST,
                   preferred_element_type=jnp.float32)


def _stage_a(x, w1, acat):
    def body(x_ref, w_ref, a_ref, h_ref, t_ref):
        h = _dot(x_ref[...], w_ref[...])
        h_ref[0] = h[:, 0:32]
        h_ref[1] = h[:, 32:64]
        t_ref[...] = _dot(h, a_ref[...])

    return pl.pallas_call(
        body,
        grid=(10,),
        in_specs=[
            pl.BlockSpec((1000, D), lambda i: (i, 0)),
            pl.BlockSpec((D, HC), lambda i: (0, 0)),
            pl.BlockSpec((HC, 32), lambda i: (0, 0)),
        ],
        out_specs=[
            pl.BlockSpec((2, 1000, 32), lambda i: (0, i, 0)),
            pl.BlockSpec((1000, 32), lambda i: (i, 0)),
        ],
        out_shape=[
            jax.ShapeDtypeStruct((2, NP, 32), jnp.float32),
            jax.ShapeDtypeStruct((NP, 32), jnp.float32),
        ],
    )(x, w1, acat)


def _stage_c(tab, den_part):
    def body(t_ref, d_ref, o_ref):
        den = jnp.sum(d_ref[...], axis=0)
        o_ref[...] = jnp.concatenate(
            [t_ref[:, 8:16], -jnp.log(den + EPS),
             jnp.zeros((t_ref.shape[0], 16), jnp.float32)], axis=1)

    return pl.pallas_call(
        body,
        grid=(10,),
        in_specs=[
            pl.BlockSpec((1024, 32), lambda i: (i, 0)),
            pl.BlockSpec((NWORK, 1024, 8), lambda i: (0, i, 0)),
        ],
        out_specs=pl.BlockSpec((1024, 32), lambda i: (i, 0)),
        out_shape=jax.ShapeDtypeStruct((NP, 32), jnp.float32),
    )(tab, den_part)


def _stage_e(part1, b1, w2p, a2w):
    def body(p_ref, b_ref, w_ref, a_ref, h_ref, t_ref):
        o = jnp.concatenate([p_ref[0], p_ref[1]], axis=1) + b_ref[...]
        x2 = jnp.where(o > 0, o, jnp.exp(o) - 1.0)
        h2 = _dot(x2, w_ref[...])
        h_ref[0] = h2[:, 0:32]
        h_ref[1] = h2[:, 32:64]
        t_ref[...] = _dot(h2, a_ref[...])

    return pl.pallas_call(
        body,
        grid=(10,),
        in_specs=[
            pl.BlockSpec((2, 1024, 32), lambda i: (0, i, 0)),
            pl.BlockSpec((1, HC), lambda i: (0, 0)),
            pl.BlockSpec((HC, NCP), lambda i: (0, 0)),
            pl.BlockSpec((NCP, 32), lambda i: (0, 0)),
        ],
        out_specs=[
            pl.BlockSpec((2, 1024, 32), lambda i: (0, i, 0)),
            pl.BlockSpec((1024, 32), lambda i: (i, 0)),
        ],
        out_shape=[
            jax.ShapeDtypeStruct((2, NP, 32), jnp.float32),
            jax.ShapeDtypeStruct((NP, 32), jnp.float32),
        ],
    )(part1, b1, w2p, a2w)


def _stage_f(part2, b2):
    def body(p_ref, b_ref, o_ref):
        o = jnp.concatenate([p_ref[0], p_ref[1]], axis=1)
        o_ref[...] = o[:, 0:NC] + b_ref[...]

    return pl.pallas_call(
        body,
        grid=(10,),
        in_specs=[
            pl.BlockSpec((2, 1000, 32), lambda i: (0, i, 0)),
            pl.BlockSpec((1, NC), lambda i: (0, 0)),
        ],
        out_specs=pl.BlockSpec((1000, NC), lambda i: (i, 0)),
        out_shape=jax.ShapeDtypeStruct((N, NC), jnp.float32),
    )(part2, b2)


CH1 = 400
CH2 = 800


def _sc_p1(ei, tab):
    ew = ei.shape[1] // NWORK

    @functools.partial(
        pl.kernel,
        out_type=jax.ShapeDtypeStruct((NWORK, NP, 8), jnp.float32),
        mesh=_mesh(),
        compiler_params=_sc_params,
        scratch_types=[
            pltpu.VMEM((CH1, 32), jnp.float32),
            pltpu.VMEM((CH1, 32), jnp.float32),
            pltpu.VMEM((NP, 8), jnp.float32),
            pltpu.VMEM((CH1,), jnp.int32),
            pltpu.VMEM((CH1,), jnp.int32),
        ],
    )
    def k(ei_h, tab_h, den_h, sbuf, dbuf, dloc, sidx, didx):
        c = lax.axis_index("core")
        s = lax.axis_index("subcore")
        w = c * NSUB + s
        iota = _lane_iota()
        lo = iota < 8
        swap = iota ^ 8
        col = iota & 7
        row2 = iota >> 3
        zf = jnp.zeros((L,), jnp.float32)

        @pl.loop(0, NP // 2)
        def _(r):
            plsc.store_scatter(dloc, [2 * r + row2, col], zf)

        @pl.loop(0, ew // CH1)
        def _(kk):
            base = w * ew + kk * CH1
            pltpu.sync_copy(ei_h.at[0].at[pl.ds(base, CH1)], sidx)
            pltpu.sync_copy(ei_h.at[1].at[pl.ds(base, CH1)], didx)
            pltpu.sync_copy(tab_h.at[sidx], sbuf)
            pltpu.sync_copy(tab_h.at[didx], dbuf)

            @pl.loop(0, CH1)
            def _(e):
                sv = sbuf[e, pl.ds(0, L)]
                dv = dbuf[e, pl.ds(0, L)]
                v = jnp.where(lo, sv, dv)
                e16 = v + _vgather(v, swap)
                lr = jnp.where(e16 >= 0, e16, 0.2 * e16)
                ex = jnp.exp(lr)
                dvec = plsc.load_gather(didx, [jnp.full((L,), e, jnp.int32)])
                plsc.addupdate_scatter(dloc, [dvec, col], ex, mask=lo)

        pltpu.sync_copy(dloc, den_h.at[w])

    return k(ei, tab)


def _sc_p2(ei, tab, dtab, hsplit):
    es = ei.shape[1] // NSUB

    @functools.partial(
        pl.kernel,
        out_type=jax.ShapeDtypeStruct((2, NP, 32), jnp.float32),
        mesh=_mesh(),
        compiler_params=_sc_params,
        scratch_types=[
            pltpu.VMEM((CH2, 32), jnp.float32),
            pltpu.VMEM((CH2, 32), jnp.float32),
            pltpu.VMEM((CH2, 32), jnp.float32),
            pltpu.VMEM((CH2,), jnp.int32),
            pltpu.VMEM((CH2,), jnp.int32),
            pltpu.VMEM_SHARED((NP, 32), jnp.float32),
        ],
    )
    def k(ei_h, tab_h, dtab_h, h_h, out_h, sbuf, dbuf, hbuf, sidx, didx, out_sh):
        c = lax.axis_index("core")
        s = lax.axis_index("subcore")
        iota = _lane_iota()
        lo = iota < 8
        swap = iota ^ 8
        hilane = iota >> 3

        @pl.loop(0, 160)
        def _(r):
            hbuf[r, pl.ds(0, L)] = jnp.zeros((L,), jnp.float32)
            hbuf[r, pl.ds(L, L)] = jnp.zeros((L,), jnp.float32)
        for m in range(4):
            pltpu.sync_copy(
                hbuf.at[pl.ds(0, 160)],
                out_sh.at[pl.ds(s * ROWS_PER_SUB + m * 160, 160)])
        plsc.subcore_barrier()

        @pl.loop(0, es // CH2)
        def _(kk):
            base = s * es + kk * CH2
            pltpu.sync_copy(ei_h.at[0].at[pl.ds(base, CH2)], sidx)
            pltpu.sync_copy(ei_h.at[1].at[pl.ds(base, CH2)], didx)
            pltpu.sync_copy(tab_h.at[sidx], sbuf)
            pltpu.sync_copy(dtab_h.at[didx], dbuf)
            pltpu.sync_copy(h_h.at[c].at[sidx], hbuf)

            @pl.loop(0, CH2)
            def _(e):
                sv = sbuf[e, pl.ds(0, L)]
                dv = dbuf[e, pl.ds(0, L)]
                v = jnp.where(lo, sv, 0.0) + dv
                lr = jnp.where(v >= 0, v, 0.2 * v)
                wv = jnp.where(lo, lr, v)
                ex = jnp.exp(wv)
                al = ex * _vgather(ex, swap)
                for q in range(2):
                    pat = hilane + 4 * c + 2 * q
                    hv = hbuf[e, pl.ds(q * L, L)]
                    hbuf[e, pl.ds(q * L, L)] = hv * _vgather(al, pat)

            pltpu.sync_copy(hbuf, out_sh.at[didx], add=True)

        plsc.subcore_barrier()
        rows = pl.ds(s * ROWS_PER_SUB, ROWS_PER_SUB)
        pltpu.sync_copy(out_sh.at[rows], out_h.at[c].at[rows])

    return k(ei, tab, dtab, hsplit)


def kernel(x, edge_index, W1, att_src1, att_dst1, bias1,
           W2, att_src2, att_dst2, bias2):
    eye = jnp.eye(H, dtype=jnp.float32)
    a_src_m = (att_src1[:, :, None] * eye[:, None, :]).reshape(HC, H)
    a_dst_m = (att_dst1[:, :, None] * eye[:, None, :]).reshape(HC, H)
    acat = jnp.pad(jnp.concatenate([a_src_m, a_dst_m], axis=1),
                   ((0, 0), (0, 16)))
    w2p = jnp.pad(W2, ((0, 0), (0, NCP - NC)))
    a2s = jnp.pad(att_src2[0], (0, NCP - NC))
    a2d = jnp.pad(att_dst2[0], (0, NCP - NC))
    a2w = jnp.pad(jnp.stack([a2s] * 8 + [a2d] * 8, axis=1),
                  ((0, 0), (0, 16)))
    b1r = bias1.reshape(1, HC)
    b2r = bias2.reshape(1, NC)

    h1, t1 = _stage_a(x, W1, acat)
    den1 = _sc_p1(edge_index, t1)
    dtab1 = _stage_c(t1, den1)
    part1 = _sc_p2(edge_index, t1, dtab1, h1)
    h2, t2 = _stage_e(part1, b1r, w2p, a2w)
    den2 = _sc_p1(edge_index, t2)
    dtab2 = _stage_c(t2, den2)
    part2 = _sc_p2(edge_index, t2, dtab2, h2)
    return _stage_f(part2, b2r)
